# baseline (speedup 1.0000x reference)
import functools

import jax
import jax.numpy as jnp
from jax import lax
from jax.experimental import pallas as pl
from jax.experimental.pallas import tpu as pltpu
from jax.experimental.pallas import tpu_sc as plsc

BATCH = 16384
EMBED = 128
FIELD0 = 100000
NC = 2
NS = 16
NW = NC * NS
BW = BATCH // NW
CH = 128
NCHUNK = BW // CH
LANES = 16
GROUPS = CH // LANES

_mesh = plsc.VectorSubcoreMesh(core_axis_name="c", subcore_axis_name="s")


@functools.partial(
    pl.kernel,
    mesh=_mesh,
    out_type=jax.ShapeDtypeStruct((BATCH,), jnp.float32),
    scratch_types=[
        pltpu.VMEM((2 * BW,), jnp.int32),
        pltpu.VMEM((4 * CH, EMBED), jnp.float32),
        pltpu.VMEM((BW,), jnp.float32),
        pltpu.VMEM((1024,), jnp.float32),
        pltpu.SemaphoreType.DMA,
        pltpu.SemaphoreType.DMA,
    ],
)
def _cmf_fwd(idx_hbm, table_hbm, out_hbm, idx_v, rows, ov, shf, sem0, sem1):
    wid = lax.axis_index("s") * NC + lax.axis_index("c")
    base = wid * BW
    pltpu.sync_copy(idx_hbm.at[pl.ds(2 * base, 2 * BW)], idx_v)

    lanes_iota = lax.iota(jnp.int32, LANES)
    bit_masks = [((lanes_iota >> k) & 1) == 1 for k in range(4)]

    def hshift(x, s, center):
        shf[pl.ds(center, LANES)] = x
        return shf[pl.ds(center - s, LANES)]

    def merge(lo, hi, k, m):
        hi2 = hi + hshift(hi, 1 << k, 64 * m + 16)
        lo2 = lo + hshift(lo, -(1 << k), 64 * m + 40)
        return jnp.where(bit_masks[k], hi2, lo2)

    def fire(c, slot_off, sem):
        pltpu.async_copy(
            table_hbm.at[idx_v.at[pl.ds(2 * CH * c, 2 * CH)]],
            rows.at[pl.ds(slot_off, 2 * CH)], sem)

    def drain(slot_off, sem):
        pltpu.make_async_copy(
            table_hbm.at[pl.ds(0, 2 * CH)],
            rows.at[pl.ds(slot_off, 2 * CH)], sem).wait()

    fire(0, 0, sem0)

    def chunk_body(c, carry):
        slot = c % 2
        off = slot * (2 * CH)

        @pl.when(slot == 0)
        def _():
            drain(0, sem0)

        @pl.when(slot == 1)
        def _():
            drain(2 * CH, sem1)

        @pl.when(jnp.logical_and(c + 1 < NCHUNK, slot == 0))
        def _():
            fire(c + 1, 2 * CH, sem1)

        @pl.when(jnp.logical_and(c + 1 < NCHUNK, slot == 1))
        def _():
            fire(c + 1, 0, sem0)

        def group(g, _):
            vecs = []
            for e in range(LANES):
                b2 = off + 2 * (g * LANES + e)
                acc = rows[b2, pl.ds(0, LANES)] * rows[b2 + 1, pl.ds(0, LANES)]
                for dj in range(1, EMBED // LANES):
                    acc = acc + (rows[b2, pl.ds(dj * LANES, LANES)]
                                 * rows[b2 + 1, pl.ds(dj * LANES, LANES)])
                vecs.append(acc)
            m = 0
            for k in range(4):
                nxt = []
                for i in range(len(vecs) // 2):
                    nxt.append(merge(vecs[2 * i], vecs[2 * i + 1], k, m))
                    m += 1
                vecs = nxt
            ov[pl.ds(c * CH + g * LANES, LANES)] = vecs[0]
            return _

        lax.fori_loop(0, GROUPS, group, 0)
        return carry

    lax.fori_loop(0, NCHUNK, chunk_body, 0)

    def sig(j, _):
        z = ov[pl.ds(j * LANES, LANES)]
        ov[pl.ds(j * LANES, LANES)] = 1.0 / (1.0 + jnp.exp(-z))
        return _

    lax.fori_loop(0, BW // LANES, sig, 0)
    pltpu.sync_copy(ov, out_hbm.at[pl.ds(base, BW)])


def kernel(x, table):
    x = x.astype(jnp.int32)
    idx = (x + jnp.array([0, FIELD0], jnp.int32)[None, :]).reshape(-1)
    return _cmf_fwd(idx, table)


# speedup vs baseline: 1.2338x; 1.2338x over previous
import functools

import jax
import jax.numpy as jnp
from jax import lax
from jax.experimental import pallas as pl
from jax.experimental.pallas import tpu as pltpu
from jax.experimental.pallas import tpu_sc as plsc

BATCH = 16384
EMBED = 128
FIELD0 = 100000
NC = 2
NS = 16
NW = NC * NS
BW = BATCH // NW
CH = 64
NCHUNK = BW // CH
NSLOT = 4
LANES = 16
GROUPS = CH // LANES

_mesh = plsc.VectorSubcoreMesh(core_axis_name="c", subcore_axis_name="s")


@functools.partial(
    pl.kernel,
    mesh=_mesh,
    out_type=jax.ShapeDtypeStruct((BATCH,), jnp.float32),
    scratch_types=[
        pltpu.VMEM((BW,), jnp.int32),
        pltpu.VMEM((BW,), jnp.int32),
        pltpu.VMEM((NSLOT * 2 * CH, EMBED), jnp.float32),
        pltpu.VMEM((BW,), jnp.float32),
        pltpu.VMEM((1024,), jnp.float32),
    ] + [pltpu.SemaphoreType.DMA for _ in range(NSLOT)],
)
def _cmf_fwd(iu_hbm, ii_hbm, table_hbm, out_hbm,
             iu_v, ii_v, rows, ov, shf, *sems):
    wid = lax.axis_index("s") * NC + lax.axis_index("c")
    base = wid * BW
    pltpu.sync_copy(iu_hbm.at[pl.ds(base, BW)], iu_v)
    pltpu.sync_copy(ii_hbm.at[pl.ds(base, BW)], ii_v)

    lanes_iota = lax.iota(jnp.int32, LANES)
    bit_masks = [((lanes_iota >> k) & 1) == 1 for k in range(4)]

    def hshift(x, s, center):
        shf[pl.ds(center, LANES)] = x
        return shf[pl.ds(center - s, LANES)]

    def merge(lo, hi, k, m):
        hi2 = hi + hshift(hi, 1 << k, 64 * m + 16)
        lo2 = lo + hshift(lo, -(1 << k), 64 * m + 40)
        return jnp.where(bit_masks[k], hi2, lo2)

    def fire(c, slot):
        off = slot * 2 * CH
        pltpu.async_copy(
            table_hbm.at[iu_v.at[pl.ds(c * CH, CH)]],
            rows.at[pl.ds(off, CH)], sems[slot])
        pltpu.async_copy(
            table_hbm.at[ii_v.at[pl.ds(c * CH, CH)]],
            rows.at[pl.ds(off + CH, CH)], sems[slot])

    def drain(slot):
        off = slot * 2 * CH
        pltpu.make_async_copy(
            table_hbm.at[pl.ds(0, CH)],
            rows.at[pl.ds(off, CH)], sems[slot]).wait()
        pltpu.make_async_copy(
            table_hbm.at[pl.ds(0, CH)],
            rows.at[pl.ds(off + CH, CH)], sems[slot]).wait()

    for c in range(NSLOT - 1):
        fire(c, c)

    def chunk_body(c, carry):
        slot = c % NSLOT
        off = slot * (2 * CH)

        for s in range(NSLOT):
            @pl.when(slot == s)
            def _(s=s):
                drain(s)

            @pl.when(jnp.logical_and(c + NSLOT - 1 < NCHUNK, slot == s))
            def _(s=s):
                fire(c + NSLOT - 1, (s + NSLOT - 1) % NSLOT)

        def group(g, gcarry):
            vecs = []
            for e in range(LANES):
                bu = off + g * LANES + e
                bv = bu + CH
                acc = rows[bu, pl.ds(0, LANES)] * rows[bv, pl.ds(0, LANES)]
                for dj in range(1, EMBED // LANES):
                    acc = acc + (rows[bu, pl.ds(dj * LANES, LANES)]
                                 * rows[bv, pl.ds(dj * LANES, LANES)])
                vecs.append(acc)
            m = 0
            for k in range(4):
                nxt = []
                for i in range(len(vecs) // 2):
                    nxt.append(merge(vecs[2 * i], vecs[2 * i + 1], k, m))
                    m += 1
                vecs = nxt
            ov[pl.ds(c * CH + g * LANES, LANES)] = vecs[0]
            return gcarry

        lax.fori_loop(0, GROUPS, group, 0)
        return carry

    lax.fori_loop(0, NCHUNK, chunk_body, 0)

    def sig(j, scarry):
        z = ov[pl.ds(j * LANES, LANES)]
        ov[pl.ds(j * LANES, LANES)] = 1.0 / (1.0 + jnp.exp(-z))
        return scarry

    lax.fori_loop(0, BW // LANES, sig, 0)
    pltpu.sync_copy(ov, out_hbm.at[pl.ds(base, BW)])


def kernel(x, table):
    x = x.astype(jnp.int32)
    iu = x[:, 0]
    ii = x[:, 1] + jnp.int32(FIELD0)
    return _cmf_fwd(iu, ii, table)


# speedup vs baseline: 1.2709x; 1.0301x over previous
import functools

import jax
import jax.numpy as jnp
from jax import lax
from jax.experimental import pallas as pl
from jax.experimental.pallas import tpu as pltpu
from jax.experimental.pallas import tpu_sc as plsc

BATCH = 16384
EMBED = 128
FIELD0 = 100000
NC = 2
NS = 16
NW = NC * NS
BW = BATCH // NW
CH = 128
NCHUNK = BW // CH
NSLOT = 2
LANES = 16
GROUPS = CH // LANES

_mesh = plsc.VectorSubcoreMesh(core_axis_name="c", subcore_axis_name="s")


@functools.partial(
    pl.kernel,
    mesh=_mesh,
    out_type=jax.ShapeDtypeStruct((BATCH,), jnp.float32),
    scratch_types=[
        pltpu.VMEM((BW,), jnp.int32),
        pltpu.VMEM((BW,), jnp.int32),
        pltpu.VMEM((NSLOT * 2 * CH, EMBED), jnp.float32),
        pltpu.VMEM((BW,), jnp.float32),
        pltpu.VMEM((1024,), jnp.float32),
    ] + [pltpu.SemaphoreType.DMA for _ in range(NSLOT)],
)
def _cmf_fwd(iu_hbm, ii_hbm, table_hbm, out_hbm, iu_v, ii_v, rows, ov, shf, *sems):
    wid = lax.axis_index("s") * NC + lax.axis_index("c")
    base = wid * BW
    pltpu.sync_copy(iu_hbm.at[pl.ds(base, BW)], iu_v)
    pltpu.sync_copy(ii_hbm.at[pl.ds(base, BW)], ii_v)

    lanes_iota = lax.iota(jnp.int32, LANES)
    bit_masks = [((lanes_iota >> k) & 1) == 1 for k in range(4)]

    def hshift(x, s, center):
        shf[pl.ds(center, LANES)] = x
        return shf[pl.ds(center - s, LANES)]

    def merge(lo, hi, k, m):
        hi2 = hi + hshift(hi, 1 << k, 64 * m + 16)
        lo2 = lo + hshift(lo, -(1 << k), 64 * m + 40)
        return jnp.where(bit_masks[k], hi2, lo2)

    def fire(c, slot):
        off = slot * 2 * CH
        pltpu.async_copy(
            table_hbm.at[iu_v.at[pl.ds(c * CH, CH)]],
            rows.at[pl.ds(off, CH)], sems[slot])
        pltpu.async_copy(
            table_hbm.at[ii_v.at[pl.ds(c * CH, CH)]],
            rows.at[pl.ds(off + CH, CH)], sems[slot])

    def drain(slot):
        off = slot * 2 * CH
        pltpu.make_async_copy(
            table_hbm.at[pl.ds(0, CH)],
            rows.at[pl.ds(off, CH)], sems[slot]).wait()
        pltpu.make_async_copy(
            table_hbm.at[pl.ds(0, CH)],
            rows.at[pl.ds(off + CH, CH)], sems[slot]).wait()

    fire(0, 0)

    def chunk_body(c, carry):
        slot = c % NSLOT
        off = slot * (2 * CH)

        for s in range(NSLOT):
            @pl.when(slot == s)
            def _(s=s):
                drain(s)

            @pl.when(jnp.logical_and(c + 1 < NCHUNK, slot == s))
            def _(s=s):
                fire(c + 1, (s + 1) % NSLOT)

        def group(g, gcarry):
            vecs = []
            for e in range(LANES):
                bu = off + g * LANES + e
                bv = bu + CH
                acc = rows[bu, pl.ds(0, LANES)] * rows[bv, pl.ds(0, LANES)]
                for dj in range(1, EMBED // LANES):
                    acc = acc + (rows[bu, pl.ds(dj * LANES, LANES)]
                                 * rows[bv, pl.ds(dj * LANES, LANES)])
                vecs.append(acc)
            m = 0
            for k in range(4):
                nxt = []
                for i in range(len(vecs) // 2):
                    nxt.append(merge(vecs[2 * i], vecs[2 * i + 1], k, m))
                    m += 1
                vecs = nxt
            ov[pl.ds(c * CH + g * LANES, LANES)] = vecs[0]
            return gcarry

        lax.fori_loop(0, GROUPS, group, 0)
        return carry

    lax.fori_loop(0, NCHUNK, chunk_body, 0)

    def sig(j, scarry):
        z = ov[pl.ds(j * LANES, LANES)]
        ov[pl.ds(j * LANES, LANES)] = 1.0 / (1.0 + jnp.exp(-z))
        return scarry

    lax.fori_loop(0, BW // LANES, sig, 0)
    pltpu.sync_copy(ov, out_hbm.at[pl.ds(base, BW)])


def kernel(x, table):
    x = x.astype(jnp.int32)
    iu = x[:, 0]
    ii = x[:, 1] + jnp.int32(FIELD0)
    return _cmf_fwd(iu, ii, table)
